# baseline (speedup 1.0000x reference)
import functools

import jax
import jax.numpy as jnp
from jax import lax
from jax.experimental import pallas as pl
from jax.experimental.pallas import tpu as pltpu
from jax.experimental.pallas import tpu_sc as plsc

B, S, D, H, E = 2, 2048, 1024, 1024, 8
N = B * S
P = 2 * N
TM = 256
NP = P + E * TM
NT = NP // TM

D2 = D // 2
H2 = H // 2

NW = 32
TW = N // NW
CH = 16


def _lane_cumsum(v):
    r = v.shape[0]
    k = 1
    while k < N:
        sh = jnp.concatenate(
            [jnp.zeros((r, k), jnp.int32), v[:, : N - k]], axis=1)
        v = v + sh
        k *= 2
    return v


def _pack_rows(x, half):
    xb = x.astype(jnp.bfloat16).astype(jnp.float32)
    bits = lax.bitcast_convert_type(xb, jnp.int32)
    lo = lax.shift_right_logical(bits[:, :half], 16)
    hi = bits[:, half:]
    return lo | hi


def _unpack_rows(p):
    lo = lax.bitcast_convert_type(lax.shift_left(p, 16), jnp.float32)
    hi = lax.bitcast_convert_type(p & jnp.int32(-65536), jnp.float32)
    return jnp.concatenate([lo, hi], axis=1)


def _route_body(x_ref, gw_ref, gb_ref, w_out, pos_out, te_out, xp_out):
    logits = jnp.dot(x_ref[...], gw_ref[...],
                     preferred_element_type=jnp.float32)
    i8 = (lax.broadcasted_iota(jnp.int32, (E, E), 0)
          == lax.broadcasted_iota(jnp.int32, (E, E), 1)).astype(jnp.float32)
    lt = lax.dot_general(i8, logits, (((1,), (1,)), ((), ())),
                         preferred_element_type=jnp.float32,
                         precision=lax.Precision.HIGHEST)
    lt = lt + gb_ref[...]
    m = jnp.max(lt, axis=0, keepdims=True)
    p = jnp.exp(lt - m)
    p = p / jnp.sum(p, axis=0, keepdims=True)
    si = lax.broadcasted_iota(jnp.int32, (E, N), 0)
    w1 = jnp.max(p, axis=0, keepdims=True)
    e0 = jnp.min(jnp.where(p == w1, si, E), axis=0, keepdims=True)
    p2 = jnp.where(si == e0, -1.0, p)
    w2 = jnp.max(p2, axis=0, keepdims=True)
    e1 = jnp.min(jnp.where(p2 == w2, si, E), axis=0, keepdims=True)
    w_out[0:1, :] = w1
    w_out[1:2, :] = w2

    ind_all = (e0 == si).astype(jnp.int32) + (e1 == si).astype(jnp.int32)
    sexc = _lane_cumsum(ind_all) - ind_all
    totals = jnp.sum(ind_all, axis=1, keepdims=True)
    padded = (totals + TM - 1) // TM * TM
    pos0 = jnp.zeros((1, N), jnp.int32)
    pos1 = jnp.zeros((1, N), jnp.int32)
    po = jnp.zeros((1, 1), jnp.int32)
    po_list = []
    for e in range(E):
        po_list.append(po)
        pos0 = pos0 + jnp.where(e0 == e, sexc[e:e + 1, :] + po, 0)
        pos1 = pos1 + jnp.where(e1 == e, sexc[e:e + 1, :] + po, 0)
        po = po + padded[e:e + 1, :]
    pos_out[0:1, :] = pos0
    pos_out[1:2, :] = pos1

    tiles = lax.broadcasted_iota(jnp.int32, (1, NT), 1) * TM
    te = jnp.zeros((1, NT), jnp.int32)
    for e in range(1, E):
        te = te + (tiles >= po_list[e]).astype(jnp.int32)
    n_active = po // TM
    te_out[...] = jnp.concatenate([te, n_active], axis=1)

    xp_out[...] = _pack_rows(x_ref[...], D2)


def _route(xf, gate_w, gate_b, interpret=False):
    return pl.pallas_call(
        _route_body,
        out_shape=[
            jax.ShapeDtypeStruct((2, N), jnp.float32),
            jax.ShapeDtypeStruct((2, N), jnp.int32),
            jax.ShapeDtypeStruct((1, NT + 1), jnp.int32),
            jax.ShapeDtypeStruct((N, D2), jnp.int32),
        ],
        interpret=interpret,
    )(xf, gate_w, gate_b.reshape(E, 1))


def _gmm_body(te_ref, xs_ref, ew_ref, eb_ref, y_ref):
    t = pl.program_id(0)

    @pl.when(t < te_ref[NT])
    def _():
        xb = _unpack_rows(xs_ref[...]).astype(jnp.bfloat16)
        y = (jnp.dot(xb, ew_ref[0], preferred_element_type=jnp.float32)
             + eb_ref[0])
        y_ref[...] = _pack_rows(y, H2)


def _gmm(te, xs, expert_w, expert_b, interpret=False):
    return pl.pallas_call(
        _gmm_body,
        grid_spec=pltpu.PrefetchScalarGridSpec(
            num_scalar_prefetch=1,
            grid=(NT,),
            in_specs=[
                pl.BlockSpec((TM, D2), lambda t, te: (t, 0)),
                pl.BlockSpec((1, D, H), lambda t, te: (te[t], 0, 0)),
                pl.BlockSpec((1, 1, H), lambda t, te: (te[t], 0, 0)),
            ],
            out_specs=pl.BlockSpec((TM, H2), lambda t, te: (t, 0)),
        ),
        out_shape=jax.ShapeDtypeStruct((NP, H2), jnp.int32),
        interpret=interpret,
    )(te, xs, expert_w.astype(jnp.bfloat16), expert_b.reshape(E, 1, H))


@functools.lru_cache(maxsize=None)
def _sc_kernels():
    mesh = plsc.VectorSubcoreMesh(core_axis_name="c", subcore_axis_name="s")

    NCH = TW // CH

    @functools.partial(
        pl.kernel,
        mesh=mesh,
        out_type=jax.ShapeDtypeStruct((NP, D2), jnp.int32),
        scratch_types=[
            pltpu.VMEM((TW,), jnp.int32),
            pltpu.VMEM((TW,), jnp.int32),
            pltpu.VMEM((CH, D2), jnp.int32),
            pltpu.VMEM((CH, D2), jnp.int32),
            pltpu.SemaphoreType.DMA,
            pltpu.SemaphoreType.DMA,
            pltpu.SemaphoreType.DMA,
            pltpu.SemaphoreType.DMA,
        ],
    )
    def sc_scatter(x_hbm, pos_hbm, xs_hbm, p0_v, p1_v, xb0, xb1,
                   rs0, rs1, ws0, ws1):
        wid = lax.axis_index("s") * 2 + lax.axis_index("c")
        tbase = wid * TW
        pltpu.sync_copy(pos_hbm.at[0, pl.ds(tbase, TW)], p0_v)
        pltpu.sync_copy(pos_hbm.at[1, pl.ds(tbase, TW)], p1_v)
        bufs = (xb0, xb1)
        rsem = (rs0, rs1)
        wsem = (ws0, ws1)
        reads = [None] * NCH
        writes = [None] * NCH
        reads[0] = pltpu.async_copy(
            x_hbm.at[pl.ds(tbase, CH)], bufs[0], rsem[0])
        for c in range(NCH):
            b = c % 2
            if c + 1 < NCH:
                if c >= 1:
                    writes[c - 1][0].wait()
                    writes[c - 1][1].wait()
                reads[c + 1] = pltpu.async_copy(
                    x_hbm.at[pl.ds(tbase + (c + 1) * CH, CH)],
                    bufs[1 - b], rsem[1 - b])
            reads[c].wait()
            idx0 = p0_v[pl.ds(c * CH, CH)]
            idx1 = p1_v[pl.ds(c * CH, CH)]
            writes[c] = (
                pltpu.async_copy(bufs[b], xs_hbm.at[idx0], wsem[b]),
                pltpu.async_copy(bufs[b], xs_hbm.at[idx1], wsem[b]),
            )
        writes[NCH - 2][0].wait()
        writes[NCH - 2][1].wait()
        writes[NCH - 1][0].wait()
        writes[NCH - 1][1].wait()

    @functools.partial(
        pl.kernel,
        mesh=mesh,
        out_type=jax.ShapeDtypeStruct((P, H2), jnp.int32),
        scratch_types=[
            pltpu.VMEM((TW,), jnp.int32),
            pltpu.VMEM((TW,), jnp.int32),
            pltpu.VMEM((CH, H2), jnp.int32),
            pltpu.VMEM((CH, H2), jnp.int32),
            pltpu.VMEM((CH, H2), jnp.int32),
            pltpu.VMEM((CH, H2), jnp.int32),
            pltpu.SemaphoreType.DMA,
            pltpu.SemaphoreType.DMA,
            pltpu.SemaphoreType.DMA,
            pltpu.SemaphoreType.DMA,
        ],
    )
    def sc_gather(ys_hbm, pos_hbm, out_hbm, p0_v, p1_v,
                  b0a, b0b, b1a, b1b, gs0, gs1, ss0, ss1):
        wid = lax.axis_index("s") * 2 + lax.axis_index("c")
        tbase = wid * TW
        pltpu.sync_copy(pos_hbm.at[0, pl.ds(tbase, TW)], p0_v)
        pltpu.sync_copy(pos_hbm.at[1, pl.ds(tbase, TW)], p1_v)
        buf0 = (b0a, b0b)
        buf1 = (b1a, b1b)
        gsem = (gs0, gs1)
        ssem = (ss0, ss1)

        def issue_reads(c):
            b = c % 2
            idx0 = p0_v[pl.ds(c * CH, CH)]
            idx1 = p1_v[pl.ds(c * CH, CH)]
            return (pltpu.async_copy(ys_hbm.at[idx0], buf0[b], gsem[b]),
                    pltpu.async_copy(ys_hbm.at[idx1], buf1[b], gsem[b]))

        reads = [None] * NCH
        writes = [None] * NCH
        reads[0] = issue_reads(0)
        for c in range(NCH):
            b = c % 2
            if c + 1 < NCH:
                if c >= 1:
                    writes[c - 1][0].wait()
                    writes[c - 1][1].wait()
                reads[c + 1] = issue_reads(c + 1)
            reads[c][0].wait()
            reads[c][1].wait()
            iot = lax.iota(jnp.int32, CH)
            dest0 = (tbase + c * CH + iot) * 2
            dest1 = dest0 + 1
            writes[c] = (
                pltpu.async_copy(buf0[b], out_hbm.at[dest0], ssem[b]),
                pltpu.async_copy(buf1[b], out_hbm.at[dest1], ssem[b]),
            )
        writes[NCH - 2][0].wait()
        writes[NCH - 2][1].wait()
        writes[NCH - 1][0].wait()
        writes[NCH - 1][1].wait()

    return sc_scatter, sc_gather


@jax.jit
def _moe(x, gate_w, gate_b, expert_w, expert_b):
    xf = x.reshape(N, D)
    sc_scatter, sc_gather = _sc_kernels()
    w2n, pos, te, xp = _route(xf, gate_w, gate_b)
    xs = sc_scatter(xp, pos)
    ys = _gmm(te.reshape(NT + 1), xs, expert_w, expert_b)
    yp = sc_gather(ys, pos)
    yb = _unpack_rows(yp)
    top2_w = w2n.T.reshape(B, S, 2)
    top2_y = yb.reshape(B, S, 2, H)
    return top2_w, top2_y


def kernel(x, gate_w, gate_b, expert_w, expert_b):
    return _moe(x, gate_w, gate_b, expert_w, expert_b)


# speedup vs baseline: 1.1318x; 1.1318x over previous
import functools

import jax
import jax.numpy as jnp
from jax import lax
from jax.experimental import pallas as pl
from jax.experimental.pallas import tpu as pltpu
from jax.experimental.pallas import tpu_sc as plsc

B, S, D, H, E = 2, 2048, 1024, 1024, 8
N = B * S
P = 2 * N
TM = 256
NP = P + E * TM
NT = NP // TM

D2 = D // 2
H2 = H // 2

NW = 32
TW = N // NW
CH = 16


def _lane_cumsum(v):
    r = v.shape[0]
    k = 1
    while k < N:
        sh = jnp.concatenate(
            [jnp.zeros((r, k), jnp.int32), v[:, : N - k]], axis=1)
        v = v + sh
        k *= 2
    return v


def _pack_rows(x, half):
    xb = x.astype(jnp.bfloat16).astype(jnp.float32)
    bits = lax.bitcast_convert_type(xb, jnp.int32)
    lo = lax.shift_right_logical(bits[:, :half], 16)
    hi = bits[:, half:]
    return lo | hi


def _unpack_rows(p):
    lo = lax.bitcast_convert_type(lax.shift_left(p, 16), jnp.float32)
    hi = lax.bitcast_convert_type(p & jnp.int32(-65536), jnp.float32)
    return jnp.concatenate([lo, hi], axis=1)


def _route_body(x_ref, gw_ref, gb_ref, w_out, pos_out, te_out, xp_out):
    logits = jnp.dot(x_ref[...], gw_ref[...],
                     preferred_element_type=jnp.float32)
    i8 = (lax.broadcasted_iota(jnp.int32, (E, E), 0)
          == lax.broadcasted_iota(jnp.int32, (E, E), 1)).astype(jnp.float32)
    lt = lax.dot_general(i8, logits, (((1,), (1,)), ((), ())),
                         preferred_element_type=jnp.float32,
                         precision=lax.Precision.HIGHEST)
    lt = lt + gb_ref[...]
    m = jnp.max(lt, axis=0, keepdims=True)
    p = jnp.exp(lt - m)
    p = p / jnp.sum(p, axis=0, keepdims=True)
    si = lax.broadcasted_iota(jnp.int32, (E, N), 0)
    w1 = jnp.max(p, axis=0, keepdims=True)
    e0 = jnp.min(jnp.where(p == w1, si, E), axis=0, keepdims=True)
    p2 = jnp.where(si == e0, -1.0, p)
    w2 = jnp.max(p2, axis=0, keepdims=True)
    e1 = jnp.min(jnp.where(p2 == w2, si, E), axis=0, keepdims=True)
    w_out[0:1, :] = w1
    w_out[1:2, :] = w2

    ind_all = (e0 == si).astype(jnp.int32) + (e1 == si).astype(jnp.int32)
    sexc = _lane_cumsum(ind_all) - ind_all
    totals = jnp.sum(ind_all, axis=1, keepdims=True)
    padded = (totals + TM - 1) // TM * TM
    pos0 = jnp.zeros((1, N), jnp.int32)
    pos1 = jnp.zeros((1, N), jnp.int32)
    po = jnp.zeros((1, 1), jnp.int32)
    po_list = []
    for e in range(E):
        po_list.append(po)
        pos0 = pos0 + jnp.where(e0 == e, sexc[e:e + 1, :] + po, 0)
        pos1 = pos1 + jnp.where(e1 == e, sexc[e:e + 1, :] + po, 0)
        po = po + padded[e:e + 1, :]
    pos_out[0:1, :] = pos0
    pos_out[1:2, :] = pos1

    tiles = lax.broadcasted_iota(jnp.int32, (1, NT), 1) * TM
    te = jnp.zeros((1, NT), jnp.int32)
    for e in range(1, E):
        te = te + (tiles >= po_list[e]).astype(jnp.int32)
    n_active = po // TM
    te_out[...] = jnp.concatenate([te, n_active], axis=1)

    xp_out[...] = _pack_rows(x_ref[...], D2)


def _route(xf, gate_w, gate_b, interpret=False):
    return pl.pallas_call(
        _route_body,
        out_shape=[
            jax.ShapeDtypeStruct((2, N), jnp.float32),
            jax.ShapeDtypeStruct((2, N), jnp.int32),
            jax.ShapeDtypeStruct((1, NT + 1), jnp.int32),
            jax.ShapeDtypeStruct((N, D2), jnp.int32),
        ],
        interpret=interpret,
    )(xf, gate_w, gate_b.reshape(E, 1))


def _gmm_body(te_ref, xs_ref, ew_ref, eb_ref, y_ref):
    t = pl.program_id(0)

    @pl.when(t < te_ref[NT])
    def _():
        xb = _unpack_rows(xs_ref[...]).astype(jnp.bfloat16)
        y_ref[...] = (jnp.dot(xb, ew_ref[0],
                              preferred_element_type=jnp.float32) + eb_ref[0])


def _gmm(te, xs, expert_w, expert_b, interpret=False):
    return pl.pallas_call(
        _gmm_body,
        grid_spec=pltpu.PrefetchScalarGridSpec(
            num_scalar_prefetch=1,
            grid=(NT,),
            in_specs=[
                pl.BlockSpec((TM, D2), lambda t, te: (t, 0)),
                pl.BlockSpec((1, D, H), lambda t, te: (te[t], 0, 0)),
                pl.BlockSpec((1, 1, H), lambda t, te: (te[t], 0, 0)),
            ],
            out_specs=pl.BlockSpec((TM, H), lambda t, te: (t, 0)),
        ),
        out_shape=jax.ShapeDtypeStruct((NP, H), jnp.float32),
        interpret=interpret,
    )(te, xs, expert_w.astype(jnp.bfloat16), expert_b.reshape(E, 1, H))


@functools.lru_cache(maxsize=None)
def _sc_kernels():
    mesh = plsc.VectorSubcoreMesh(core_axis_name="c", subcore_axis_name="s")

    NCH = TW // CH

    @functools.partial(
        pl.kernel,
        mesh=mesh,
        out_type=jax.ShapeDtypeStruct((NP, D2), jnp.int32),
        scratch_types=[
            pltpu.VMEM((TW,), jnp.int32),
            pltpu.VMEM((TW,), jnp.int32),
            pltpu.VMEM((CH, D2), jnp.int32),
            pltpu.VMEM((CH, D2), jnp.int32),
            pltpu.SemaphoreType.DMA,
            pltpu.SemaphoreType.DMA,
            pltpu.SemaphoreType.DMA,
            pltpu.SemaphoreType.DMA,
        ],
    )
    def sc_scatter(x_hbm, pos_hbm, xs_hbm, p0_v, p1_v, xb0, xb1,
                   rs0, rs1, ws0, ws1):
        wid = lax.axis_index("s") * 2 + lax.axis_index("c")
        tbase = wid * TW
        pltpu.sync_copy(pos_hbm.at[0, pl.ds(tbase, TW)], p0_v)
        pltpu.sync_copy(pos_hbm.at[1, pl.ds(tbase, TW)], p1_v)
        bufs = (xb0, xb1)
        rsem = (rs0, rs1)
        wsem = (ws0, ws1)
        reads = [None] * NCH
        writes = [None] * NCH
        reads[0] = pltpu.async_copy(
            x_hbm.at[pl.ds(tbase, CH)], bufs[0], rsem[0])
        for c in range(NCH):
            b = c % 2
            if c + 1 < NCH:
                if c >= 1:
                    writes[c - 1][0].wait()
                    writes[c - 1][1].wait()
                reads[c + 1] = pltpu.async_copy(
                    x_hbm.at[pl.ds(tbase + (c + 1) * CH, CH)],
                    bufs[1 - b], rsem[1 - b])
            reads[c].wait()
            idx0 = p0_v[pl.ds(c * CH, CH)]
            idx1 = p1_v[pl.ds(c * CH, CH)]
            writes[c] = (
                pltpu.async_copy(bufs[b], xs_hbm.at[idx0], wsem[b]),
                pltpu.async_copy(bufs[b], xs_hbm.at[idx1], wsem[b]),
            )
        writes[NCH - 2][0].wait()
        writes[NCH - 2][1].wait()
        writes[NCH - 1][0].wait()
        writes[NCH - 1][1].wait()

    @functools.partial(
        pl.kernel,
        mesh=mesh,
        out_type=jax.ShapeDtypeStruct((P, H), jnp.float32),
        scratch_types=[
            pltpu.VMEM((TW,), jnp.int32),
            pltpu.VMEM((TW,), jnp.int32),
            pltpu.VMEM((CH, H), jnp.float32),
            pltpu.VMEM((CH, H), jnp.float32),
            pltpu.VMEM((CH, H), jnp.float32),
            pltpu.VMEM((CH, H), jnp.float32),
            pltpu.SemaphoreType.DMA,
            pltpu.SemaphoreType.DMA,
            pltpu.SemaphoreType.DMA,
            pltpu.SemaphoreType.DMA,
        ],
    )
    def sc_gather(ys_hbm, pos_hbm, out_hbm, p0_v, p1_v,
                  b0a, b0b, b1a, b1b, gs0, gs1, ss0, ss1):
        wid = lax.axis_index("s") * 2 + lax.axis_index("c")
        tbase = wid * TW
        pltpu.sync_copy(pos_hbm.at[0, pl.ds(tbase, TW)], p0_v)
        pltpu.sync_copy(pos_hbm.at[1, pl.ds(tbase, TW)], p1_v)
        buf0 = (b0a, b0b)
        buf1 = (b1a, b1b)
        gsem = (gs0, gs1)
        ssem = (ss0, ss1)

        def issue_reads(c):
            b = c % 2
            idx0 = p0_v[pl.ds(c * CH, CH)]
            idx1 = p1_v[pl.ds(c * CH, CH)]
            return (pltpu.async_copy(ys_hbm.at[idx0], buf0[b], gsem[b]),
                    pltpu.async_copy(ys_hbm.at[idx1], buf1[b], gsem[b]))

        reads = [None] * NCH
        writes = [None] * NCH
        reads[0] = issue_reads(0)
        for c in range(NCH):
            b = c % 2
            if c + 1 < NCH:
                if c >= 1:
                    writes[c - 1][0].wait()
                    writes[c - 1][1].wait()
                reads[c + 1] = issue_reads(c + 1)
            reads[c][0].wait()
            reads[c][1].wait()
            iot = lax.iota(jnp.int32, CH)
            dest0 = (tbase + c * CH + iot) * 2
            dest1 = dest0 + 1
            writes[c] = (
                pltpu.async_copy(buf0[b], out_hbm.at[dest0], ssem[b]),
                pltpu.async_copy(buf1[b], out_hbm.at[dest1], ssem[b]),
            )
        writes[NCH - 2][0].wait()
        writes[NCH - 2][1].wait()
        writes[NCH - 1][0].wait()
        writes[NCH - 1][1].wait()

    return sc_scatter, sc_gather


@jax.jit
def _moe(x, gate_w, gate_b, expert_w, expert_b):
    xf = x.reshape(N, D)
    sc_scatter, sc_gather = _sc_kernels()
    w2n, pos, te, xp = _route(xf, gate_w, gate_b)
    xs = sc_scatter(xp, pos)
    ys = _gmm(te.reshape(NT + 1), xs, expert_w, expert_b)
    yout = sc_gather(ys, pos)
    top2_w = w2n.T.reshape(B, S, 2)
    top2_y = yout.reshape(B, S, 2, H)
    return top2_w, top2_y


def kernel(x, gate_w, gate_b, expert_w, expert_b):
    return _moe(x, gate_w, gate_b, expert_w, expert_b)


# speedup vs baseline: 1.1337x; 1.0017x over previous
import functools

import jax
import jax.numpy as jnp
from jax import lax
from jax.experimental import pallas as pl
from jax.experimental.pallas import tpu as pltpu
from jax.experimental.pallas import tpu_sc as plsc

B, S, D, H, E = 2, 2048, 1024, 1024, 8
N = B * S
P = 2 * N
TM = 256
NP = P + E * TM
NT = NP // TM

D2 = D // 2
H2 = H // 2

NW = 32
TW = N // NW
CH = 16


def _lane_cumsum(v):
    r = v.shape[0]
    k = 1
    while k < N:
        sh = jnp.concatenate(
            [jnp.zeros((r, k), jnp.int32), v[:, : N - k]], axis=1)
        v = v + sh
        k *= 2
    return v


def _pack_rows(x, half):
    xb = x.astype(jnp.bfloat16).astype(jnp.float32)
    bits = lax.bitcast_convert_type(xb, jnp.int32)
    lo = lax.shift_right_logical(bits[:, :half], 16)
    hi = bits[:, half:]
    return lo | hi


def _unpack_rows(p):
    lo = lax.bitcast_convert_type(lax.shift_left(p, 16), jnp.float32)
    hi = lax.bitcast_convert_type(p & jnp.int32(-65536), jnp.float32)
    return jnp.concatenate([lo, hi], axis=1)


def _route_body(x_ref, gw_ref, gb_ref, w_out, pos_out, te_out, xp_out):
    logits = jnp.dot(x_ref[...], gw_ref[...],
                     preferred_element_type=jnp.float32)
    i8 = (lax.broadcasted_iota(jnp.int32, (E, E), 0)
          == lax.broadcasted_iota(jnp.int32, (E, E), 1)).astype(jnp.float32)
    lt = lax.dot_general(i8, logits, (((1,), (1,)), ((), ())),
                         preferred_element_type=jnp.float32,
                         precision=lax.Precision.HIGHEST)
    lt = lt + gb_ref[...]
    m = jnp.max(lt, axis=0, keepdims=True)
    p = jnp.exp(lt - m)
    p = p / jnp.sum(p, axis=0, keepdims=True)
    si = lax.broadcasted_iota(jnp.int32, (E, N), 0)
    w1 = jnp.max(p, axis=0, keepdims=True)
    e0 = jnp.min(jnp.where(p == w1, si, E), axis=0, keepdims=True)
    p2 = jnp.where(si == e0, -1.0, p)
    w2 = jnp.max(p2, axis=0, keepdims=True)
    e1 = jnp.min(jnp.where(p2 == w2, si, E), axis=0, keepdims=True)
    w_out[0:1, :] = w1
    w_out[1:2, :] = w2

    ind_all = (e0 == si).astype(jnp.int32) + (e1 == si).astype(jnp.int32)
    sexc = _lane_cumsum(ind_all) - ind_all
    totals = jnp.sum(ind_all, axis=1, keepdims=True)
    padded = (totals + TM - 1) // TM * TM
    pos0 = jnp.zeros((1, N), jnp.int32)
    pos1 = jnp.zeros((1, N), jnp.int32)
    po = jnp.zeros((1, 1), jnp.int32)
    po_list = []
    for e in range(E):
        po_list.append(po)
        pos0 = pos0 + jnp.where(e0 == e, sexc[e:e + 1, :] + po, 0)
        pos1 = pos1 + jnp.where(e1 == e, sexc[e:e + 1, :] + po, 0)
        po = po + padded[e:e + 1, :]
    pos_out[0:1, :] = pos0
    pos_out[1:2, :] = pos1

    tiles = lax.broadcasted_iota(jnp.int32, (1, NT), 1) * TM
    te = jnp.zeros((1, NT), jnp.int32)
    for e in range(1, E):
        te = te + (tiles >= po_list[e]).astype(jnp.int32)
    n_active = po // TM
    te_out[...] = jnp.concatenate([te, n_active], axis=1)

    xp_out[...] = _pack_rows(x_ref[...], D2)


def _route(xf, gate_w, gate_b, interpret=False):
    return pl.pallas_call(
        _route_body,
        out_shape=[
            jax.ShapeDtypeStruct((2, N), jnp.float32),
            jax.ShapeDtypeStruct((2, N), jnp.int32),
            jax.ShapeDtypeStruct((1, NT + 1), jnp.int32),
            jax.ShapeDtypeStruct((N, D2), jnp.int32),
        ],
        interpret=interpret,
    )(xf, gate_w, gate_b.reshape(E, 1))


def _gmm_body(te_ref, xs_ref, ew_ref, eb_ref, y_ref):
    t = pl.program_id(0)

    @pl.when(t < te_ref[NT])
    def _():
        p = xs_ref[...]
        lo = lax.bitcast_convert_type(
            lax.shift_left(p, 16), jnp.float32).astype(jnp.bfloat16)
        hi = lax.bitcast_convert_type(
            p & jnp.int32(-65536), jnp.float32).astype(jnp.bfloat16)
        w = ew_ref[0]
        y = jnp.dot(lo, w[:D2], preferred_element_type=jnp.float32)
        y = y + jnp.dot(hi, w[D2:], preferred_element_type=jnp.float32)
        y_ref[...] = y + eb_ref[0]


def _gmm(te, xs, expert_w, expert_b, interpret=False):
    return pl.pallas_call(
        _gmm_body,
        grid_spec=pltpu.PrefetchScalarGridSpec(
            num_scalar_prefetch=1,
            grid=(NT,),
            in_specs=[
                pl.BlockSpec((TM, D2), lambda t, te: (t, 0)),
                pl.BlockSpec((1, D, H), lambda t, te: (te[t], 0, 0)),
                pl.BlockSpec((1, 1, H), lambda t, te: (te[t], 0, 0)),
            ],
            out_specs=pl.BlockSpec((TM, H), lambda t, te: (t, 0)),
        ),
        out_shape=jax.ShapeDtypeStruct((NP, H), jnp.float32),
        interpret=interpret,
    )(te, xs, expert_w.astype(jnp.bfloat16), expert_b.reshape(E, 1, H))


@functools.lru_cache(maxsize=None)
def _sc_kernels():
    mesh = plsc.VectorSubcoreMesh(core_axis_name="c", subcore_axis_name="s")

    NCH = TW // CH

    @functools.partial(
        pl.kernel,
        mesh=mesh,
        out_type=jax.ShapeDtypeStruct((NP, D2), jnp.int32),
        scratch_types=[
            pltpu.VMEM((TW,), jnp.int32),
            pltpu.VMEM((TW,), jnp.int32),
            pltpu.VMEM((CH, D2), jnp.int32),
            pltpu.VMEM((CH, D2), jnp.int32),
            pltpu.SemaphoreType.DMA,
            pltpu.SemaphoreType.DMA,
            pltpu.SemaphoreType.DMA,
            pltpu.SemaphoreType.DMA,
        ],
    )
    def sc_scatter(x_hbm, pos_hbm, xs_hbm, p0_v, p1_v, xb0, xb1,
                   rs0, rs1, ws0, ws1):
        wid = lax.axis_index("s") * 2 + lax.axis_index("c")
        tbase = wid * TW
        pltpu.sync_copy(pos_hbm.at[0, pl.ds(tbase, TW)], p0_v)
        pltpu.sync_copy(pos_hbm.at[1, pl.ds(tbase, TW)], p1_v)
        bufs = (xb0, xb1)
        rsem = (rs0, rs1)
        wsem = (ws0, ws1)
        reads = [None] * NCH
        writes = [None] * NCH
        reads[0] = pltpu.async_copy(
            x_hbm.at[pl.ds(tbase, CH)], bufs[0], rsem[0])
        for c in range(NCH):
            b = c % 2
            if c + 1 < NCH:
                if c >= 1:
                    writes[c - 1][0].wait()
                    writes[c - 1][1].wait()
                reads[c + 1] = pltpu.async_copy(
                    x_hbm.at[pl.ds(tbase + (c + 1) * CH, CH)],
                    bufs[1 - b], rsem[1 - b])
            reads[c].wait()
            idx0 = p0_v[pl.ds(c * CH, CH)]
            idx1 = p1_v[pl.ds(c * CH, CH)]
            writes[c] = (
                pltpu.async_copy(bufs[b], xs_hbm.at[idx0], wsem[b]),
                pltpu.async_copy(bufs[b], xs_hbm.at[idx1], wsem[b]),
            )
        writes[NCH - 2][0].wait()
        writes[NCH - 2][1].wait()
        writes[NCH - 1][0].wait()
        writes[NCH - 1][1].wait()

    @functools.partial(
        pl.kernel,
        mesh=mesh,
        out_type=jax.ShapeDtypeStruct((P, H), jnp.float32),
        scratch_types=[
            pltpu.VMEM((TW,), jnp.int32),
            pltpu.VMEM((TW,), jnp.int32),
            pltpu.VMEM((CH, H), jnp.float32),
            pltpu.VMEM((CH, H), jnp.float32),
            pltpu.VMEM((CH, H), jnp.float32),
            pltpu.VMEM((CH, H), jnp.float32),
            pltpu.SemaphoreType.DMA,
            pltpu.SemaphoreType.DMA,
            pltpu.SemaphoreType.DMA,
            pltpu.SemaphoreType.DMA,
        ],
    )
    def sc_gather(ys_hbm, pos_hbm, out_hbm, p0_v, p1_v,
                  b0a, b0b, b1a, b1b, gs0, gs1, ss0, ss1):
        wid = lax.axis_index("s") * 2 + lax.axis_index("c")
        tbase = wid * TW
        pltpu.sync_copy(pos_hbm.at[0, pl.ds(tbase, TW)], p0_v)
        pltpu.sync_copy(pos_hbm.at[1, pl.ds(tbase, TW)], p1_v)
        buf0 = (b0a, b0b)
        buf1 = (b1a, b1b)
        gsem = (gs0, gs1)
        ssem = (ss0, ss1)

        def issue_reads(c):
            b = c % 2
            idx0 = p0_v[pl.ds(c * CH, CH)]
            idx1 = p1_v[pl.ds(c * CH, CH)]
            return (pltpu.async_copy(ys_hbm.at[idx0], buf0[b], gsem[b]),
                    pltpu.async_copy(ys_hbm.at[idx1], buf1[b], gsem[b]))

        reads = [None] * NCH
        writes = [None] * NCH
        reads[0] = issue_reads(0)
        for c in range(NCH):
            b = c % 2
            if c + 1 < NCH:
                if c >= 1:
                    writes[c - 1][0].wait()
                    writes[c - 1][1].wait()
                reads[c + 1] = issue_reads(c + 1)
            reads[c][0].wait()
            reads[c][1].wait()
            iot = lax.iota(jnp.int32, CH)
            dest0 = (tbase + c * CH + iot) * 2
            dest1 = dest0 + 1
            writes[c] = (
                pltpu.async_copy(buf0[b], out_hbm.at[dest0], ssem[b]),
                pltpu.async_copy(buf1[b], out_hbm.at[dest1], ssem[b]),
            )
        writes[NCH - 2][0].wait()
        writes[NCH - 2][1].wait()
        writes[NCH - 1][0].wait()
        writes[NCH - 1][1].wait()

    return sc_scatter, sc_gather


@jax.jit
def _moe(x, gate_w, gate_b, expert_w, expert_b):
    xf = x.reshape(N, D)
    sc_scatter, sc_gather = _sc_kernels()
    w2n, pos, te, xp = _route(xf, gate_w, gate_b)
    xs = sc_scatter(xp, pos)
    ys = _gmm(te.reshape(NT + 1), xs, expert_w, expert_b)
    yout = sc_gather(ys, pos)
    top2_w = w2n.T.reshape(B, S, 2)
    top2_y = yout.reshape(B, S, 2, H)
    return top2_w, top2_y


def kernel(x, gate_w, gate_b, expert_w, expert_b):
    return _moe(x, gate_w, gate_b, expert_w, expert_b)


# speedup vs baseline: 1.1772x; 1.0384x over previous
import functools

import jax
import jax.numpy as jnp
from jax import lax
from jax.experimental import pallas as pl
from jax.experimental.pallas import tpu as pltpu
from jax.experimental.pallas import tpu_sc as plsc

B, S, D, H, E = 2, 2048, 1024, 1024, 8
N = B * S
P = 2 * N
TM = 512
NP = P + E * TM
NT = NP // TM

D2 = D // 2
H2 = H // 2

NW = 32
TW = N // NW
CH = 16


def _lane_cumsum(v):
    r = v.shape[0]
    k = 1
    while k < N:
        sh = jnp.concatenate(
            [jnp.zeros((r, k), jnp.int32), v[:, : N - k]], axis=1)
        v = v + sh
        k *= 2
    return v


def _pack_rows(x, half):
    xb = x.astype(jnp.bfloat16).astype(jnp.float32)
    bits = lax.bitcast_convert_type(xb, jnp.int32)
    lo = lax.shift_right_logical(bits[:, :half], 16)
    hi = bits[:, half:]
    return lo | hi


def _unpack_rows(p):
    lo = lax.bitcast_convert_type(lax.shift_left(p, 16), jnp.float32)
    hi = lax.bitcast_convert_type(p & jnp.int32(-65536), jnp.float32)
    return jnp.concatenate([lo, hi], axis=1)


def _route_body(x_ref, gw_ref, gb_ref, w_out, pos_out, te_out, xp_out):
    logits = jnp.dot(x_ref[...], gw_ref[...],
                     preferred_element_type=jnp.float32)
    i8 = (lax.broadcasted_iota(jnp.int32, (E, E), 0)
          == lax.broadcasted_iota(jnp.int32, (E, E), 1)).astype(jnp.float32)
    lt = lax.dot_general(i8, logits, (((1,), (1,)), ((), ())),
                         preferred_element_type=jnp.float32,
                         precision=lax.Precision.HIGHEST)
    lt = lt + gb_ref[...]
    m = jnp.max(lt, axis=0, keepdims=True)
    p = jnp.exp(lt - m)
    p = p / jnp.sum(p, axis=0, keepdims=True)
    si = lax.broadcasted_iota(jnp.int32, (E, N), 0)
    w1 = jnp.max(p, axis=0, keepdims=True)
    e0 = jnp.min(jnp.where(p == w1, si, E), axis=0, keepdims=True)
    p2 = jnp.where(si == e0, -1.0, p)
    w2 = jnp.max(p2, axis=0, keepdims=True)
    e1 = jnp.min(jnp.where(p2 == w2, si, E), axis=0, keepdims=True)
    w_out[0:1, :] = w1
    w_out[1:2, :] = w2

    ind_all = (e0 == si).astype(jnp.int32) + (e1 == si).astype(jnp.int32)
    sexc = _lane_cumsum(ind_all) - ind_all
    totals = jnp.sum(ind_all, axis=1, keepdims=True)
    padded = (totals + TM - 1) // TM * TM
    pos0 = jnp.zeros((1, N), jnp.int32)
    pos1 = jnp.zeros((1, N), jnp.int32)
    po = jnp.zeros((1, 1), jnp.int32)
    po_list = []
    for e in range(E):
        po_list.append(po)
        pos0 = pos0 + jnp.where(e0 == e, sexc[e:e + 1, :] + po, 0)
        pos1 = pos1 + jnp.where(e1 == e, sexc[e:e + 1, :] + po, 0)
        po = po + padded[e:e + 1, :]
    pos_out[0:1, :] = pos0
    pos_out[1:2, :] = pos1

    tiles = lax.broadcasted_iota(jnp.int32, (1, NT), 1) * TM
    te = jnp.zeros((1, NT), jnp.int32)
    for e in range(1, E):
        te = te + (tiles >= po_list[e]).astype(jnp.int32)
    n_active = po // TM
    te_out[...] = jnp.concatenate([te, n_active], axis=1)

    xp_out[...] = _pack_rows(x_ref[...], D2)


def _route(xf, gate_w, gate_b, interpret=False):
    return pl.pallas_call(
        _route_body,
        out_shape=[
            jax.ShapeDtypeStruct((2, N), jnp.float32),
            jax.ShapeDtypeStruct((2, N), jnp.int32),
            jax.ShapeDtypeStruct((1, NT + 1), jnp.int32),
            jax.ShapeDtypeStruct((N, D2), jnp.int32),
        ],
        interpret=interpret,
    )(xf, gate_w, gate_b.reshape(E, 1))


def _gmm_body(te_ref, xs_ref, ew_ref, eb_ref, y_ref):
    t = pl.program_id(0)

    @pl.when(t < te_ref[NT])
    def _():
        p = xs_ref[...]
        lo = lax.bitcast_convert_type(
            lax.shift_left(p, 16), jnp.float32).astype(jnp.bfloat16)
        hi = lax.bitcast_convert_type(
            p & jnp.int32(-65536), jnp.float32).astype(jnp.bfloat16)
        w = ew_ref[0]
        y = jnp.dot(lo, w[:D2], preferred_element_type=jnp.float32)
        y = y + jnp.dot(hi, w[D2:], preferred_element_type=jnp.float32)
        y_ref[...] = y + eb_ref[0]


def _gmm(te, xs, expert_w, expert_b, interpret=False):
    return pl.pallas_call(
        _gmm_body,
        grid_spec=pltpu.PrefetchScalarGridSpec(
            num_scalar_prefetch=1,
            grid=(NT,),
            in_specs=[
                pl.BlockSpec((TM, D2), lambda t, te: (t, 0)),
                pl.BlockSpec((1, D, H), lambda t, te: (te[t], 0, 0)),
                pl.BlockSpec((1, 1, H), lambda t, te: (te[t], 0, 0)),
            ],
            out_specs=pl.BlockSpec((TM, H), lambda t, te: (t, 0)),
        ),
        out_shape=jax.ShapeDtypeStruct((NP, H), jnp.float32),
        interpret=interpret,
    )(te, xs, expert_w.astype(jnp.bfloat16), expert_b.reshape(E, 1, H))


@functools.lru_cache(maxsize=None)
def _sc_kernels():
    mesh = plsc.VectorSubcoreMesh(core_axis_name="c", subcore_axis_name="s")

    NCH = TW // CH

    @functools.partial(
        pl.kernel,
        mesh=mesh,
        out_type=jax.ShapeDtypeStruct((NP, D2), jnp.int32),
        scratch_types=[
            pltpu.VMEM((TW,), jnp.int32),
            pltpu.VMEM((TW,), jnp.int32),
            pltpu.VMEM((CH, D2), jnp.int32),
            pltpu.VMEM((CH, D2), jnp.int32),
            pltpu.SemaphoreType.DMA,
            pltpu.SemaphoreType.DMA,
            pltpu.SemaphoreType.DMA,
            pltpu.SemaphoreType.DMA,
        ],
    )
    def sc_scatter(x_hbm, pos_hbm, xs_hbm, p0_v, p1_v, xb0, xb1,
                   rs0, rs1, ws0, ws1):
        wid = lax.axis_index("s") * 2 + lax.axis_index("c")
        tbase = wid * TW
        pltpu.sync_copy(pos_hbm.at[0, pl.ds(tbase, TW)], p0_v)
        pltpu.sync_copy(pos_hbm.at[1, pl.ds(tbase, TW)], p1_v)
        bufs = (xb0, xb1)
        rsem = (rs0, rs1)
        wsem = (ws0, ws1)
        reads = [None] * NCH
        writes = [None] * NCH
        reads[0] = pltpu.async_copy(
            x_hbm.at[pl.ds(tbase, CH)], bufs[0], rsem[0])
        for c in range(NCH):
            b = c % 2
            if c + 1 < NCH:
                if c >= 1:
                    writes[c - 1][0].wait()
                    writes[c - 1][1].wait()
                reads[c + 1] = pltpu.async_copy(
                    x_hbm.at[pl.ds(tbase + (c + 1) * CH, CH)],
                    bufs[1 - b], rsem[1 - b])
            reads[c].wait()
            idx0 = p0_v[pl.ds(c * CH, CH)]
            idx1 = p1_v[pl.ds(c * CH, CH)]
            writes[c] = (
                pltpu.async_copy(bufs[b], xs_hbm.at[idx0], wsem[b]),
                pltpu.async_copy(bufs[b], xs_hbm.at[idx1], wsem[b]),
            )
        writes[NCH - 2][0].wait()
        writes[NCH - 2][1].wait()
        writes[NCH - 1][0].wait()
        writes[NCH - 1][1].wait()

    @functools.partial(
        pl.kernel,
        mesh=mesh,
        out_type=jax.ShapeDtypeStruct((P, H), jnp.float32),
        scratch_types=[
            pltpu.VMEM((TW,), jnp.int32),
            pltpu.VMEM((TW,), jnp.int32),
            pltpu.VMEM((CH, H), jnp.float32),
            pltpu.VMEM((CH, H), jnp.float32),
            pltpu.VMEM((CH, H), jnp.float32),
            pltpu.VMEM((CH, H), jnp.float32),
            pltpu.SemaphoreType.DMA,
            pltpu.SemaphoreType.DMA,
            pltpu.SemaphoreType.DMA,
            pltpu.SemaphoreType.DMA,
        ],
    )
    def sc_gather(ys_hbm, pos_hbm, out_hbm, p0_v, p1_v,
                  b0a, b0b, b1a, b1b, gs0, gs1, ss0, ss1):
        wid = lax.axis_index("s") * 2 + lax.axis_index("c")
        tbase = wid * TW
        pltpu.sync_copy(pos_hbm.at[0, pl.ds(tbase, TW)], p0_v)
        pltpu.sync_copy(pos_hbm.at[1, pl.ds(tbase, TW)], p1_v)
        buf0 = (b0a, b0b)
        buf1 = (b1a, b1b)
        gsem = (gs0, gs1)
        ssem = (ss0, ss1)

        def issue_reads(c):
            b = c % 2
            idx0 = p0_v[pl.ds(c * CH, CH)]
            idx1 = p1_v[pl.ds(c * CH, CH)]
            return (pltpu.async_copy(ys_hbm.at[idx0], buf0[b], gsem[b]),
                    pltpu.async_copy(ys_hbm.at[idx1], buf1[b], gsem[b]))

        reads = [None] * NCH
        writes = [None] * NCH
        reads[0] = issue_reads(0)
        for c in range(NCH):
            b = c % 2
            if c + 1 < NCH:
                if c >= 1:
                    writes[c - 1][0].wait()
                    writes[c - 1][1].wait()
                reads[c + 1] = issue_reads(c + 1)
            reads[c][0].wait()
            reads[c][1].wait()
            iot = lax.iota(jnp.int32, CH)
            dest0 = (tbase + c * CH + iot) * 2
            dest1 = dest0 + 1
            writes[c] = (
                pltpu.async_copy(buf0[b], out_hbm.at[dest0], ssem[b]),
                pltpu.async_copy(buf1[b], out_hbm.at[dest1], ssem[b]),
            )
        writes[NCH - 2][0].wait()
        writes[NCH - 2][1].wait()
        writes[NCH - 1][0].wait()
        writes[NCH - 1][1].wait()

    return sc_scatter, sc_gather


@jax.jit
def _moe(x, gate_w, gate_b, expert_w, expert_b):
    xf = x.reshape(N, D)
    sc_scatter, sc_gather = _sc_kernels()
    w2n, pos, te, xp = _route(xf, gate_w, gate_b)
    xs = sc_scatter(xp, pos)
    ys = _gmm(te.reshape(NT + 1), xs, expert_w, expert_b)
    yout = sc_gather(ys, pos)
    top2_w = w2n.T.reshape(B, S, 2)
    top2_y = yout.reshape(B, S, 2, H)
    return top2_w, top2_y


def kernel(x, gate_w, gate_b, expert_w, expert_b):
    return _moe(x, gate_w, gate_b, expert_w, expert_b)


# speedup vs baseline: 1.1776x; 1.0003x over previous
import functools

import jax
import jax.numpy as jnp
from jax import lax
from jax.experimental import pallas as pl
from jax.experimental.pallas import tpu as pltpu
from jax.experimental.pallas import tpu_sc as plsc

B, S, D, H, E = 2, 2048, 1024, 1024, 8
N = B * S
P = 2 * N
TM = 512
NP = P + E * TM
NT = NP // TM

D2 = D // 2
H2 = H // 2

NW = 32
TW = N // NW
CH = 16


def _lane_cumsum(v):
    r = v.shape[0]
    k = 1
    while k < N:
        sh = jnp.concatenate(
            [jnp.zeros((r, k), jnp.int32), v[:, : N - k]], axis=1)
        v = v + sh
        k *= 2
    return v


def _pack_rows(x, half):
    xb = x.astype(jnp.bfloat16).astype(jnp.float32)
    bits = lax.bitcast_convert_type(xb, jnp.int32)
    lo = lax.shift_right_logical(bits[:, :half], 16)
    hi = bits[:, half:]
    return lo | hi


def _unpack_rows(p):
    lo = lax.bitcast_convert_type(lax.shift_left(p, 16), jnp.float32)
    hi = lax.bitcast_convert_type(p & jnp.int32(-65536), jnp.float32)
    return jnp.concatenate([lo, hi], axis=1)


def _route_body(x_ref, gw_ref, gb_ref, w_out, pos_out, te_out, xp_out):
    logits = jnp.dot(x_ref[...], gw_ref[...],
                     preferred_element_type=jnp.float32)
    i8 = (lax.broadcasted_iota(jnp.int32, (E, E), 0)
          == lax.broadcasted_iota(jnp.int32, (E, E), 1)).astype(jnp.float32)
    lt = lax.dot_general(i8, logits, (((1,), (1,)), ((), ())),
                         preferred_element_type=jnp.float32,
                         precision=lax.Precision.HIGHEST)
    lt = lt + gb_ref[...]
    m = jnp.max(lt, axis=0, keepdims=True)
    p = jnp.exp(lt - m)
    p = p / jnp.sum(p, axis=0, keepdims=True)
    si = lax.broadcasted_iota(jnp.int32, (E, N), 0)
    w1 = jnp.max(p, axis=0, keepdims=True)
    e0 = jnp.min(jnp.where(p == w1, si, E), axis=0, keepdims=True)
    p2 = jnp.where(si == e0, -1.0, p)
    w2 = jnp.max(p2, axis=0, keepdims=True)
    e1 = jnp.min(jnp.where(p2 == w2, si, E), axis=0, keepdims=True)
    w_out[0:1, :] = w1
    w_out[1:2, :] = w2

    ind_all = (e0 == si).astype(jnp.int32) + (e1 == si).astype(jnp.int32)
    sexc = _lane_cumsum(ind_all) - ind_all
    totals = jnp.sum(ind_all, axis=1, keepdims=True)
    padded = (totals + TM - 1) // TM * TM
    pos0 = jnp.zeros((1, N), jnp.int32)
    pos1 = jnp.zeros((1, N), jnp.int32)
    po = jnp.zeros((1, 1), jnp.int32)
    po_list = []
    for e in range(E):
        po_list.append(po)
        pos0 = pos0 + jnp.where(e0 == e, sexc[e:e + 1, :] + po, 0)
        pos1 = pos1 + jnp.where(e1 == e, sexc[e:e + 1, :] + po, 0)
        po = po + padded[e:e + 1, :]
    pos_out[0:1, :] = pos0
    pos_out[1:2, :] = pos1

    tiles = lax.broadcasted_iota(jnp.int32, (1, NT), 1) * TM
    te = jnp.zeros((1, NT), jnp.int32)
    for e in range(1, E):
        te = te + (tiles >= po_list[e]).astype(jnp.int32)
    n_active = po // TM
    te_out[...] = jnp.concatenate([te, n_active], axis=1)

    xp_out[...] = _pack_rows(x_ref[...], D2)


def _route(xf, gate_w, gate_b, interpret=False):
    return pl.pallas_call(
        _route_body,
        out_shape=[
            jax.ShapeDtypeStruct((2, N), jnp.float32),
            jax.ShapeDtypeStruct((2, N), jnp.int32),
            jax.ShapeDtypeStruct((1, NT + 1), jnp.int32),
            jax.ShapeDtypeStruct((N, D2), jnp.int32),
        ],
        interpret=interpret,
    )(xf, gate_w, gate_b.reshape(E, 1))


def _gmm_body(te_ref, xs_ref, ew_ref, eb_ref, y_ref):
    t = pl.program_id(0)

    @pl.when(t < te_ref[NT])
    def _():
        p = xs_ref[...]
        lo = lax.bitcast_convert_type(
            lax.shift_left(p, 16), jnp.float32).astype(jnp.bfloat16)
        hi = lax.bitcast_convert_type(
            p & jnp.int32(-65536), jnp.float32).astype(jnp.bfloat16)
        w = ew_ref[0]
        y = jnp.dot(lo, w[:D2], preferred_element_type=jnp.float32)
        y = y + jnp.dot(hi, w[D2:], preferred_element_type=jnp.float32)
        y_ref[...] = y + eb_ref[0]


def _gmm(te, xs, expert_w, expert_b, interpret=False):
    return pl.pallas_call(
        _gmm_body,
        grid_spec=pltpu.PrefetchScalarGridSpec(
            num_scalar_prefetch=1,
            grid=(NT,),
            in_specs=[
                pl.BlockSpec((TM, D2), lambda t, te: (t, 0)),
                pl.BlockSpec((1, D, H), lambda t, te: (te[t], 0, 0)),
                pl.BlockSpec((1, 1, H), lambda t, te: (te[t], 0, 0)),
            ],
            out_specs=pl.BlockSpec((TM, H), lambda t, te: (t, 0)),
        ),
        out_shape=jax.ShapeDtypeStruct((NP, H), jnp.float32),
        interpret=interpret,
    )(te, xs, expert_w.astype(jnp.bfloat16), expert_b.reshape(E, 1, H))


@functools.lru_cache(maxsize=None)
def _sc_kernels():
    mesh = plsc.VectorSubcoreMesh(core_axis_name="c", subcore_axis_name="s")

    NCH = TW // CH

    @functools.partial(
        pl.kernel,
        mesh=mesh,
        out_type=jax.ShapeDtypeStruct((NP, D2), jnp.int32),
        scratch_types=(
            [pltpu.VMEM((TW,), jnp.int32)] * 2
            + [pltpu.VMEM((CH, D2), jnp.int32)] * 4
            + [pltpu.SemaphoreType.DMA] * 8
        ),
    )
    def sc_scatter(x_hbm, pos_hbm, xs_hbm, p0_v, p1_v, *bs):
        bufs, rsem, wsem = bs[:4], bs[4:8], bs[8:12]
        nb = 4
        wid = lax.axis_index("s") * 2 + lax.axis_index("c")
        tbase = wid * TW
        pltpu.sync_copy(pos_hbm.at[0, pl.ds(tbase, TW)], p0_v)
        pltpu.sync_copy(pos_hbm.at[1, pl.ds(tbase, TW)], p1_v)
        reads = [None] * NCH
        writes = [None] * NCH
        reads[0] = pltpu.async_copy(
            x_hbm.at[pl.ds(tbase, CH)], bufs[0], rsem[0])
        for c in range(NCH):
            b = c % nb
            if c + 1 < NCH:
                if c + 1 - nb >= 0:
                    writes[c + 1 - nb][0].wait()
                    writes[c + 1 - nb][1].wait()
                reads[c + 1] = pltpu.async_copy(
                    x_hbm.at[pl.ds(tbase + (c + 1) * CH, CH)],
                    bufs[(c + 1) % nb], rsem[(c + 1) % nb])
            reads[c].wait()
            idx0 = p0_v[pl.ds(c * CH, CH)]
            idx1 = p1_v[pl.ds(c * CH, CH)]
            writes[c] = (
                pltpu.async_copy(bufs[b], xs_hbm.at[idx0], wsem[b]),
                pltpu.async_copy(bufs[b], xs_hbm.at[idx1], wsem[b]),
            )
        for k in range(max(0, NCH - nb), NCH):
            writes[k][0].wait()
            writes[k][1].wait()

    @functools.partial(
        pl.kernel,
        mesh=mesh,
        out_type=jax.ShapeDtypeStruct((P, H), jnp.float32),
        scratch_types=(
            [pltpu.VMEM((TW,), jnp.int32)] * 2
            + [pltpu.VMEM((CH, H), jnp.float32)] * 6
            + [pltpu.SemaphoreType.DMA] * 6
        ),
    )
    def sc_gather(ys_hbm, pos_hbm, out_hbm, p0_v, p1_v, *bs):
        buf0, buf1 = bs[:3], bs[3:6]
        gsem, ssem = bs[6:9], bs[9:12]
        nb = 3
        wid = lax.axis_index("s") * 2 + lax.axis_index("c")
        tbase = wid * TW
        pltpu.sync_copy(pos_hbm.at[0, pl.ds(tbase, TW)], p0_v)
        pltpu.sync_copy(pos_hbm.at[1, pl.ds(tbase, TW)], p1_v)

        def issue_reads(c):
            b = c % nb
            idx0 = p0_v[pl.ds(c * CH, CH)]
            idx1 = p1_v[pl.ds(c * CH, CH)]
            return (pltpu.async_copy(ys_hbm.at[idx0], buf0[b], gsem[b]),
                    pltpu.async_copy(ys_hbm.at[idx1], buf1[b], gsem[b]))

        reads = [None] * NCH
        writes = [None] * NCH
        reads[0] = issue_reads(0)
        for c in range(NCH):
            b = c % nb
            if c + 1 < NCH:
                if c + 1 - nb >= 0:
                    writes[c + 1 - nb][0].wait()
                    writes[c + 1 - nb][1].wait()
                reads[c + 1] = issue_reads(c + 1)
            reads[c][0].wait()
            reads[c][1].wait()
            iot = lax.iota(jnp.int32, CH)
            dest0 = (tbase + c * CH + iot) * 2
            dest1 = dest0 + 1
            writes[c] = (
                pltpu.async_copy(buf0[b], out_hbm.at[dest0], ssem[b]),
                pltpu.async_copy(buf1[b], out_hbm.at[dest1], ssem[b]),
            )
        for k in range(max(0, NCH - nb), NCH):
            writes[k][0].wait()
            writes[k][1].wait()

    return sc_scatter, sc_gather


@jax.jit
def _moe(x, gate_w, gate_b, expert_w, expert_b):
    xf = x.reshape(N, D)
    sc_scatter, sc_gather = _sc_kernels()
    w2n, pos, te, xp = _route(xf, gate_w, gate_b)
    xs = sc_scatter(xp, pos)
    ys = _gmm(te.reshape(NT + 1), xs, expert_w, expert_b)
    yout = sc_gather(ys, pos)
    top2_w = w2n.T.reshape(B, S, 2)
    top2_y = yout.reshape(B, S, 2, H)
    return top2_w, top2_y


def kernel(x, gate_w, gate_b, expert_w, expert_b):
    return _moe(x, gate_w, gate_b, expert_w, expert_b)
